# baseline (speedup 1.0000x reference)
import math

import jax
import jax.numpy as jnp
from jax.experimental import pallas as pl
from jax.experimental.pallas import tpu as pltpu


def _round_up(x: int, m: int) -> int:
    return ((x + m - 1) // m) * m


def _mm_kernel(x_ref, w_ref, o_ref):
    o_ref[...] = jnp.dot(
        x_ref[...], w_ref[...], preferred_element_type=jnp.float32
    ).astype(o_ref.dtype)


def kernel(x, weight):
    E, H, I = weight.shape
    assert x.shape[-1] == H, "hidden dim mismatch"
    assert x.shape[0] in (1, E), "input/weight disagree on number of experts"
    if x.shape[0] == 1 and E > 1:
        x = jnp.broadcast_to(x, (E,) + x.shape[1:])

    inner = x.shape[1:-1]
    T = int(math.prod(inner)) if inner else 1
    out_dtype = jnp.result_type(x.dtype, weight.dtype)

    x2 = x.reshape(E, T, H)

    T_p = _round_up(T, 8)
    H_p = _round_up(H, 128)
    tn = 512 if I % 512 == 0 else 128
    I_p = _round_up(I, tn)

    if (T_p, H_p) != (T, H):
        x2 = jnp.pad(x2, ((0, 0), (0, T_p - T), (0, H_p - H)))
    if I_p != I:
        weight = jnp.pad(weight, ((0, 0), (0, 0), (0, I_p - I)))
    if H_p != H:
        weight = jnp.pad(weight, ((0, 0), (0, H_p - H), (0, 0)))

    grid = (E, I_p // tn)

    in_bytes = x2.size * x2.dtype.itemsize + weight.size * weight.dtype.itemsize
    out_bytes = E * T_p * I_p * jnp.dtype(out_dtype).itemsize

    out = pl.pallas_call(
        _mm_kernel,
        out_shape=jax.ShapeDtypeStruct((E, T_p, I_p), out_dtype),
        grid=grid,
        in_specs=[
            pl.BlockSpec((None, T_p, H_p), lambda e, n: (e, 0, 0)),
            pl.BlockSpec((None, H_p, tn), lambda e, n: (e, 0, n)),
        ],
        out_specs=pl.BlockSpec((None, T_p, tn), lambda e, n: (e, 0, n)),
        compiler_params=pltpu.CompilerParams(
            dimension_semantics=("parallel", "arbitrary"),
            vmem_limit_bytes=64 * 1024 * 1024,
        ),
        cost_estimate=pl.CostEstimate(
            flops=2 * E * T_p * H_p * I_p,
            transcendentals=0,
            bytes_accessed=in_bytes + out_bytes,
        ),
    )(x2, weight)

    out = out[:, :T, :I].reshape((E,) + inner + (I,))
    return out


# speedup vs baseline: 1.1877x; 1.1877x over previous
import math

import jax
import jax.numpy as jnp
from jax.experimental import pallas as pl
from jax.experimental.pallas import tpu as pltpu


def _round_up(x: int, m: int) -> int:
    return ((x + m - 1) // m) * m


def _mm_kernel(x_ref, w_ref, o_ref):
    o_ref[...] = jnp.dot(
        x_ref[...], w_ref[...], preferred_element_type=jnp.float32
    ).astype(o_ref.dtype)


def kernel(x, weight):
    E, H, I = weight.shape
    assert x.shape[-1] == H, "hidden dim mismatch"
    assert x.shape[0] in (1, E), "input/weight disagree on number of experts"
    if x.shape[0] == 1 and E > 1:
        x = jnp.broadcast_to(x, (E,) + x.shape[1:])

    inner = x.shape[1:-1]
    T = int(math.prod(inner)) if inner else 1
    out_dtype = jnp.result_type(x.dtype, weight.dtype)

    x2 = x.reshape(E, T, H)

    T_p = _round_up(T, 8)
    H_p = _round_up(H, 128)
    tn = 2048 if I % 2048 == 0 else (1024 if I % 1024 == 0 else 128)
    I_p = _round_up(I, tn)

    if (T_p, H_p) != (T, H):
        x2 = jnp.pad(x2, ((0, 0), (0, T_p - T), (0, H_p - H)))
    if I_p != I:
        weight = jnp.pad(weight, ((0, 0), (0, 0), (0, I_p - I)))
    if H_p != H:
        weight = jnp.pad(weight, ((0, 0), (0, H_p - H), (0, 0)))

    grid = (E, I_p // tn)

    in_bytes = x2.size * x2.dtype.itemsize + weight.size * weight.dtype.itemsize
    out_bytes = E * T_p * I_p * jnp.dtype(out_dtype).itemsize

    out = pl.pallas_call(
        _mm_kernel,
        out_shape=jax.ShapeDtypeStruct((E, T_p, I_p), out_dtype),
        grid=grid,
        in_specs=[
            pl.BlockSpec((None, T_p, H_p), lambda e, n: (e, 0, 0)),
            pl.BlockSpec((None, H_p, tn), lambda e, n: (e, 0, n)),
        ],
        out_specs=pl.BlockSpec((None, T_p, tn), lambda e, n: (e, 0, n)),
        compiler_params=pltpu.CompilerParams(
            dimension_semantics=("parallel", "arbitrary"),
            vmem_limit_bytes=64 * 1024 * 1024,
        ),
        cost_estimate=pl.CostEstimate(
            flops=2 * E * T_p * H_p * I_p,
            transcendentals=0,
            bytes_accessed=in_bytes + out_bytes,
        ),
    )(x2, weight)

    out = out[:, :T, :I].reshape((E,) + inner + (I,))
    return out
